# baseline (speedup 1.0000x reference)
import functools

import jax
import jax.numpy as jnp
from jax import lax
from jax.experimental import pallas as pl
from jax.experimental.pallas import tpu as pltpu
from jax.experimental.pallas import tpu_sc as plsc

NC = 2
NS = 16
NW = NC * NS
BLK = 256
NBUF = 8
G = 4


@functools.partial(jax.jit, static_argnames=("emb",))
def _emb_lookup(xf, W, *, emb):
    n_blocks = xf.shape[0]
    nb = n_blocks // NW
    mesh = plsc.VectorSubcoreMesh(
        core_axis_name="c", subcore_axis_name="s", num_cores=NC, num_subcores=NS
    )

    @functools.partial(
        pl.kernel,
        out_type=jax.ShapeDtypeStruct((n_blocks, BLK, emb), jnp.float32),
        mesh=mesh,
        scratch_types=[
            pltpu.VMEM((nb, BLK), jnp.int32),
            pltpu.VMEM((NBUF, BLK, emb), jnp.float32),
            pltpu.SemaphoreType.DMA,
            [pltpu.SemaphoreType.DMA] * NBUF,
            [pltpu.SemaphoreType.DMA] * NBUF,
        ],
        compiler_params=pltpu.CompilerParams(use_tc_tiling_on_sc=False),
    )
    def k(x_hbm, w_hbm, out_hbm, idx_v, rows_v, isem, gsems, osems):
        wid = lax.axis_index("s") * NC + lax.axis_index("c")
        blk0 = wid * nb

        def fire_gather(j, b):
            pltpu.async_copy(w_hbm.at[idx_v.at[j]], rows_v.at[b], gsems[b])

        def wait_gather(b):
            pltpu.make_async_copy(
                w_hbm.at[idx_v.at[0]], rows_v.at[b], gsems[b]
            ).wait()

        def fire_out(j, b):
            pltpu.async_copy(rows_v.at[b], out_hbm.at[blk0 + j], osems[b])

        def wait_out(b):
            pltpu.make_async_copy(
                rows_v.at[b], out_hbm.at[0], osems[b]
            ).wait()

        pltpu.async_copy(x_hbm.at[pl.ds(blk0, nb)], idx_v, isem)
        pltpu.make_async_copy(x_hbm.at[pl.ds(0, nb)], idx_v, isem).wait()

        for j in range(G):
            fire_gather(j, j)

        for j in range(0, NBUF):
            b = j % NBUF
            wait_gather(b)
            fire_out(j, b)
            if j + G <= nb - 1:
                bg = (j + G) % NBUF
                if j + G >= NBUF:
                    wait_out(bg)
                fire_gather(j + G, bg)

        n_mid = (nb - G - NBUF) // NBUF

        def body(k_, carry):
            for t in range(NBUF):
                j = NBUF * k_ + t
                b = t
                bg = (t + G) % NBUF
                wait_gather(b)
                fire_out(j, b)
                wait_out(bg)
                fire_gather(j + G, bg)
            return carry

        lax.fori_loop(1, 1 + n_mid, body, 0)

        for j in range(NBUF * (1 + n_mid), nb):
            b = j % NBUF
            wait_gather(b)
            fire_out(j, b)
            if j + G <= nb - 1:
                bg = (j + G) % NBUF
                wait_out(bg)
                fire_gather(j + G, bg)

        for j in range(nb - NBUF, nb):
            wait_out(j % NBUF)

    return k(xf, W)


def kernel(x, W):
    x2 = x if x.ndim > 1 else x.reshape(x.shape[0], 1)
    seq, batch = x2.shape
    emb = W.shape[1]
    xf = x2.astype(jnp.int32).reshape(-1, BLK)
    out = _emb_lookup(xf, W, emb=emb)
    return out.reshape(seq, batch, emb)


# speedup vs baseline: 1.0019x; 1.0019x over previous
import functools

import jax
import jax.numpy as jnp
from jax import lax
from jax.experimental import pallas as pl
from jax.experimental.pallas import tpu as pltpu
from jax.experimental.pallas import tpu_sc as plsc

NC = 2
NS = 16
NW = NC * NS
BLK = 512
NBUF = 6
G = 3


@functools.partial(jax.jit, static_argnames=("emb",))
def _emb_lookup(xf, W, *, emb):
    n_blocks = xf.shape[0]
    nb = n_blocks // NW
    mesh = plsc.VectorSubcoreMesh(
        core_axis_name="c", subcore_axis_name="s", num_cores=NC, num_subcores=NS
    )

    @functools.partial(
        pl.kernel,
        out_type=jax.ShapeDtypeStruct((n_blocks, BLK, emb), jnp.float32),
        mesh=mesh,
        scratch_types=[
            pltpu.VMEM((nb, BLK), jnp.int32),
            pltpu.VMEM((NBUF, BLK, emb), jnp.float32),
            pltpu.SemaphoreType.DMA,
            [pltpu.SemaphoreType.DMA] * NBUF,
            [pltpu.SemaphoreType.DMA] * NBUF,
        ],
        compiler_params=pltpu.CompilerParams(use_tc_tiling_on_sc=False),
    )
    def k(x_hbm, w_hbm, out_hbm, idx_v, rows_v, isem, gsems, osems):
        wid = lax.axis_index("s") * NC + lax.axis_index("c")
        blk0 = wid * nb

        def fire_gather(j, b):
            pltpu.async_copy(w_hbm.at[idx_v.at[j]], rows_v.at[b], gsems[b])

        def wait_gather(b):
            pltpu.make_async_copy(
                w_hbm.at[idx_v.at[0]], rows_v.at[b], gsems[b]
            ).wait()

        def fire_out(j, b):
            pltpu.async_copy(rows_v.at[b], out_hbm.at[blk0 + j], osems[b])

        def wait_out(b):
            pltpu.make_async_copy(
                rows_v.at[b], out_hbm.at[0], osems[b]
            ).wait()

        pltpu.async_copy(x_hbm.at[pl.ds(blk0, nb)], idx_v, isem)
        pltpu.make_async_copy(x_hbm.at[pl.ds(0, nb)], idx_v, isem).wait()

        for j in range(G):
            fire_gather(j, j)

        for j in range(0, NBUF):
            b = j % NBUF
            wait_gather(b)
            fire_out(j, b)
            if j + G <= nb - 1:
                bg = (j + G) % NBUF
                if j + G >= NBUF:
                    wait_out(bg)
                fire_gather(j + G, bg)

        n_mid = (nb - G - NBUF) // NBUF

        def body(k_, carry):
            for t in range(NBUF):
                j = NBUF * k_ + t
                b = t
                bg = (t + G) % NBUF
                wait_gather(b)
                fire_out(j, b)
                wait_out(bg)
                fire_gather(j + G, bg)
            return carry

        lax.fori_loop(1, 1 + n_mid, body, 0)

        for j in range(NBUF * (1 + n_mid), nb):
            b = j % NBUF
            wait_gather(b)
            fire_out(j, b)
            if j + G <= nb - 1:
                bg = (j + G) % NBUF
                wait_out(bg)
                fire_gather(j + G, bg)

        for j in range(nb - NBUF, nb):
            wait_out(j % NBUF)

    return k(xf, W)


def kernel(x, W):
    x2 = x if x.ndim > 1 else x.reshape(x.shape[0], 1)
    seq, batch = x2.shape
    emb = W.shape[1]
    xf = x2.astype(jnp.int32).reshape(-1, BLK)
    out = _emb_lookup(xf, W, emb=emb)
    return out.reshape(seq, batch, emb)
